# Initial kernel scaffold; baseline (speedup 1.0000x reference)
import jax
import jax.numpy as jnp
from jax.experimental import pallas as pl


def kernel(x, edge_index, W0, b0, W1, b1, Wout, bout):
    raise NotImplementedError("write your pallas kernel here")


# speedup vs baseline: 7.2164x; 7.2164x over previous
import functools

import jax
import jax.numpy as jnp
import numpy as np
from jax import lax
from jax.experimental import pallas as pl
from jax.experimental.pallas import tpu as pltpu
from jax.experimental.pallas import tpu_sc as plsc

N = 10000
NPAD = 10240
NW = 32
NODES_PER_W = NPAD // NW
WORDS = NPAD // 32
WORDS_PAD = 384
E_REAL = 320000 + N
E_PAD = 331776
EC_N = 64
NCH_N = (E_PAD // NW) // EC_N
EC_S = 128
D = 128
NCLS = 64

_mesh = plsc.VectorSubcoreMesh(
    core_axis_name="c", subcore_axis_name="s", num_cores=2, num_subcores=16)
_sc_params = pltpu.CompilerParams(needs_layout_passes=False)


def _wid():
    return lax.axis_index("s") * 2 + lax.axis_index("c")


def _popcount(w):
    c55 = jnp.int32(0x55555555)
    c33 = jnp.int32(0x33333333)
    c0f = jnp.int32(0x0F0F0F0F)
    c01 = jnp.int32(0x01010101)
    one = jnp.int32(1)
    w = w - (lax.shift_right_logical(w, one) & c55)
    w = (w & c33) + (lax.shift_right_logical(w, one + one) & c33)
    w = (w + lax.shift_right_logical(w, jnp.int32(4))) & c0f
    return lax.shift_right_logical(w * c01, jnp.int32(24))


def _hsum16(buf_ref):
    rows = lax.iota(jnp.int32, 16)
    tot = jnp.zeros((16,), jnp.int32)
    for t in range(16):
        cols = jnp.full((16,), t, jnp.int32)
        tot = tot + plsc.load_gather(buf_ref, [rows, cols])
    return tot


@functools.partial(
    pl.kernel,
    out_type=jax.ShapeDtypeStruct((NPAD,), jnp.int32),
    mesh=_mesh,
    compiler_params=_sc_params,
    scratch_types=[
        pltpu.VMEM((16, WORDS_PAD), jnp.int32),
        pltpu.VMEM((16, 16), jnp.int32),
        pltpu.VMEM((NODES_PER_W,), jnp.int32),
    ],
)
def _deg(table_hbm, deg_hbm, rows_v, tmp_v, deg_v):
    w = _wid()
    rowbase = pl.multiple_of(w * NODES_PER_W, 8)

    def chunk(i, _):
        pltpu.sync_copy(table_hbm.at[pl.ds(rowbase + i * 16, 16), :], rows_v)

        def row(j, _):
            acc = jnp.zeros((16,), jnp.int32)
            for t in range(WORDS // 16):
                acc = acc + _popcount(rows_v[j, pl.ds(t * 16, 16)])
            tmp_v[j, :] = acc
            return 0

        lax.fori_loop(0, 16, row, 0)
        deg_v[pl.ds(i * 16, 16)] = _hsum16(tmp_v)
        return 0

    lax.fori_loop(0, NODES_PER_W // 16, chunk, 0)
    pltpu.sync_copy(deg_v, deg_hbm.at[pl.ds(rowbase, NODES_PER_W)])


@functools.partial(
    pl.kernel,
    out_type=jax.ShapeDtypeStruct((E_PAD,), jnp.float32),
    mesh=_mesh,
    compiler_params=_sc_params,
    scratch_types=[
        pltpu.VMEM((EC_N,), jnp.int32),
        pltpu.VMEM((EC_N,), jnp.int32),
        pltpu.VMEM((EC_N, WORDS_PAD), jnp.int32),
        pltpu.VMEM((EC_N, WORDS_PAD), jnp.int32),
        pltpu.VMEM((NPAD,), jnp.int32),
        pltpu.VMEM((EC_N, 16), jnp.int32),
        pltpu.VMEM((EC_N,), jnp.float32),
        pltpu.SemaphoreType.DMA,
        pltpu.SemaphoreType.DMA,
    ],
)
def _norm(table_hbm, src_hbm, dst_hbm, deg_hbm, norm_hbm,
          su_v, sv_v, ur_v, vr_v, deg_v, int_v, nrm_v, sem_u, sem_v):
    w = _wid()
    ebase = w * (E_PAD // NW)
    pltpu.sync_copy(deg_hbm, deg_v)

    def chunk(k, _):
        off = pl.multiple_of(ebase + k * EC_N, 8)
        pltpu.sync_copy(src_hbm.at[pl.ds(off, EC_N)], su_v)
        pltpu.sync_copy(dst_hbm.at[pl.ds(off, EC_N)], sv_v)
        cu = pltpu.async_copy(table_hbm.at[su_v], ur_v, sem_u)
        cv = pltpu.async_copy(table_hbm.at[sv_v], vr_v, sem_v)
        cu.wait()
        cv.wait()

        def edge(j, _):
            acc = jnp.zeros((16,), jnp.int32)
            for t in range(WORDS // 16):
                uw = ur_v[j, pl.ds(t * 16, 16)]
                vw = vr_v[j, pl.ds(t * 16, 16)]
                acc = acc + _popcount(uw & vw)
            int_v[j, :] = acc
            return 0

        lax.fori_loop(0, EC_N, edge, 0)

        for g in range(EC_N // 16):
            rows = lax.iota(jnp.int32, 16) + jnp.int32(g * 16)
            it = jnp.zeros((16,), jnp.int32)
            for t in range(16):
                cols = jnp.full((16,), t, jnp.int32)
                it = it + plsc.load_gather(int_v, [rows, cols])
            iu = su_v[pl.ds(g * 16, 16)]
            iv = sv_v[pl.ds(g * 16, 16)]
            du = plsc.load_gather(deg_v, [iu])
            dv = plsc.load_gather(deg_v, [iv])
            un = du + dv - it
            nrm_v[pl.ds(g * 16, 16)] = (
                it.astype(jnp.float32) / un.astype(jnp.float32))
        pltpu.sync_copy(nrm_v, norm_hbm.at[pl.ds(off, EC_N)])
        return 0

    lax.fori_loop(0, NCH_N, chunk, 0)


@functools.partial(
    pl.kernel,
    out_type=jax.ShapeDtypeStruct((NPAD, D), jnp.float32),
    mesh=_mesh,
    compiler_params=_sc_params,
    scratch_types=[
        pltpu.VMEM((NW + 16,), jnp.int32),
        pltpu.VMEM((NW + 16,), jnp.int32),
        pltpu.VMEM((EC_S,), jnp.int32),
        pltpu.VMEM((EC_S + 16,), jnp.int32),
        pltpu.VMEM((EC_S + 16,), jnp.float32),
        pltpu.VMEM((EC_S, D), jnp.float32),
        pltpu.VMEM((NODES_PER_W, D), jnp.float32),
        pltpu.VMEM((D,), jnp.float32),
        pltpu.SemaphoreType.DMA,
    ],
)
def _segmax(h_hbm, src_hbm, dst_hbm, norm_hbm, starts_hbm, nch_hbm, b_hbm,
            out_hbm, st_v, nc_v, su_v, sd_v, nr_v, rows_v, acc_v, b_v, sem):
    w = _wid()
    base = pl.multiple_of(w * NODES_PER_W, 8)
    pltpu.sync_copy(starts_hbm, st_v.at[pl.ds(0, NW)])
    pltpu.sync_copy(nch_hbm, nc_v.at[pl.ds(0, NW)])
    pltpu.sync_copy(b_hbm, b_v)

    def initrow(r, _):
        for t in range(D // 16):
            acc_v[r, pl.ds(t * 16, 16)] = jnp.full((16,), -3e38, jnp.float32)
        return 0

    lax.fori_loop(0, NODES_PER_W, initrow, 0)

    s8 = st_v[pl.ds(w, 16)][0]
    nch = nc_v[pl.ds(w, 16)][0]

    def chunk(k, _):
        off = pl.multiple_of(jnp.minimum(s8 + k * EC_S, E_PAD - EC_S), 8)
        pltpu.sync_copy(src_hbm.at[pl.ds(off, EC_S)], su_v)
        pltpu.sync_copy(dst_hbm.at[pl.ds(off, EC_S)], sd_v.at[pl.ds(0, EC_S)])
        pltpu.sync_copy(norm_hbm.at[pl.ds(off, EC_S)], nr_v.at[pl.ds(0, EC_S)])
        pltpu.async_copy(h_hbm.at[su_v], rows_v, sem).wait()

        def edge(j, _):
            d = sd_v[pl.ds(j, 16)][0]
            dl = d - base

            @pl.when((dl >= 0) & (dl < NODES_PER_W))
            def _():
                nrm = nr_v[pl.ds(j, 16)][0]
                for t in range(D // 16):
                    msg = rows_v[j, pl.ds(t * 16, 16)] * nrm
                    cur = acc_v[dl, pl.ds(t * 16, 16)]
                    acc_v[dl, pl.ds(t * 16, 16)] = jnp.maximum(cur, msg)

            return 0

        return lax.fori_loop(0, EC_S, edge, 0)

    lax.fori_loop(0, nch, chunk, 0)

    def finrow(r, _):
        for t in range(D // 16):
            a = acc_v[r, pl.ds(t * 16, 16)] + b_v[pl.ds(t * 16, 16)]
            acc_v[r, pl.ds(t * 16, 16)] = jnp.maximum(a, 0.0)
        return 0

    lax.fori_loop(0, NODES_PER_W, finrow, 0)
    pltpu.sync_copy(acc_v, out_hbm.at[pl.ds(base, NODES_PER_W), :])


def _mm_body(x_ref, w_ref, o_ref):
    o_ref[...] = lax.dot_general(
        x_ref[...], w_ref[...], (((1,), (1,)), ((), ())),
        preferred_element_type=jnp.float32)


def _matmul_wt(x, W):
    M, K = x.shape
    Nout = W.shape[0]
    BM = 1024
    return pl.pallas_call(
        _mm_body,
        grid=(M // BM,),
        in_specs=[pl.BlockSpec((BM, K), lambda i: (i, 0)),
                  pl.BlockSpec((Nout, K), lambda i: (0, 0))],
        out_specs=pl.BlockSpec((BM, Nout), lambda i: (i, 0)),
        out_shape=jax.ShapeDtypeStruct((M, Nout), jnp.float32),
    )(x, W)


def _head_body(h_ref, w_ref, b_ref, o_ref):
    logits = lax.dot_general(
        h_ref[...], w_ref[...], (((1,), (1,)), ((), ())),
        preferred_element_type=jnp.float32) + b_ref[...]
    m = jnp.max(logits, axis=-1, keepdims=True)
    ls = logits - m
    o_ref[...] = ls - jnp.log(jnp.sum(jnp.exp(ls), axis=-1, keepdims=True))


def _head(h, Wout, bout):
    M, K = h.shape
    BM = 1024
    return pl.pallas_call(
        _head_body,
        grid=(M // BM,),
        in_specs=[pl.BlockSpec((BM, K), lambda i: (i, 0)),
                  pl.BlockSpec((NCLS, K), lambda i: (0, 0)),
                  pl.BlockSpec((1, NCLS), lambda i: (0, 0))],
        out_specs=pl.BlockSpec((BM, NCLS), lambda i: (i, 0)),
        out_shape=jax.ShapeDtypeStruct((M, NCLS), jnp.float32),
    )(h, Wout, bout.reshape(1, NCLS))


def kernel(x, edge_index, W0, b0, W1, b1, Wout, bout):
    self_loops = jnp.arange(N, dtype=jnp.int32)
    src0 = jnp.concatenate([edge_index[0].astype(jnp.int32), self_loops])
    dst0 = jnp.concatenate([edge_index[1].astype(jnp.int32), self_loops])

    adj = jnp.zeros((NPAD, NPAD), jnp.bool_)
    adj = adj.at[src0, dst0].set(True)
    adj = adj.at[dst0, src0].set(True)
    weights = jnp.uint32(1) << jnp.arange(32, dtype=jnp.uint32)
    packed = (adj.reshape(NPAD, WORDS, 32).astype(jnp.uint32) * weights).sum(
        axis=-1, dtype=jnp.uint32)
    table = lax.bitcast_convert_type(packed, jnp.int32)
    table = jnp.pad(table, ((0, 0), (0, WORDS_PAD - WORDS)))

    npad_e = E_PAD - E_REAL
    src_all = jnp.concatenate([src0, jnp.zeros((npad_e,), jnp.int32)])
    dst_all = jnp.concatenate(
        [dst0, jnp.full((npad_e,), NPAD - 1, jnp.int32)])
    order = jnp.argsort(dst_all)
    src_s = jnp.take(src_all, order).astype(jnp.int32)
    dst_s = jnp.take(dst_all, order).astype(jnp.int32)

    bnd = jnp.searchsorted(
        dst_s, jnp.arange(NW + 1, dtype=jnp.int32) * NODES_PER_W
    ).astype(jnp.int32)
    starts8 = bnd[:-1] & jnp.int32(~7)
    nch = (bnd[1:] - starts8 + (EC_S - 1)) // EC_S

    deg = _deg(table)
    norm = _norm(table, src_s, dst_s, deg)

    xp = jnp.zeros((NPAD, D), jnp.float32).at[:N].set(x)
    h = _matmul_wt(xp, W0)
    h = _segmax(h, src_s, dst_s, norm, starts8, nch, b0)
    h = _matmul_wt(h, W1)
    h = _segmax(h, src_s, dst_s, norm, starts8, nch, b1)
    out = _head(h, Wout, bout)
    return out[:N]


# speedup vs baseline: 13.9382x; 1.9315x over previous
import functools

import jax
import jax.numpy as jnp
import numpy as np
from jax import lax
from jax.experimental import pallas as pl
from jax.experimental.pallas import tpu as pltpu
from jax.experimental.pallas import tpu_sc as plsc

N = 10000
NPAD = 10240
NW = 32
NODES_PER_W = NPAD // NW
WORDS = NPAD // 32
WORDS_PAD = 384
E_REAL = 320000 + N
E_PAD = 331776
EC_N = 64
NCH_N = (E_PAD // NW) // EC_N
EC_S = 128
D = 128
NCLS = 64

_mesh = plsc.VectorSubcoreMesh(
    core_axis_name="c", subcore_axis_name="s", num_cores=2, num_subcores=16)
_sc_params = pltpu.CompilerParams(needs_layout_passes=False)


def _wid():
    return lax.axis_index("s") * 2 + lax.axis_index("c")


def _popcount(w):
    c55 = jnp.int32(0x55555555)
    c33 = jnp.int32(0x33333333)
    c0f = jnp.int32(0x0F0F0F0F)
    c01 = jnp.int32(0x01010101)
    one = jnp.int32(1)
    w = w - (lax.shift_right_logical(w, one) & c55)
    w = (w & c33) + (lax.shift_right_logical(w, one + one) & c33)
    w = (w + lax.shift_right_logical(w, jnp.int32(4))) & c0f
    return lax.shift_right_logical(w * c01, jnp.int32(24))


@functools.partial(
    pl.kernel,
    out_type=[
        jax.ShapeDtypeStruct((NPAD, WORDS_PAD), jnp.int32),
        jax.ShapeDtypeStruct((NPAD,), jnp.int32),
    ],
    mesh=_mesh,
    compiler_params=_sc_params,
    scratch_types=[
        pltpu.VMEM((NODES_PER_W, WORDS_PAD), jnp.int32),
        pltpu.VMEM((EC_S + 16,), jnp.int32),
        pltpu.VMEM((EC_S + 16,), jnp.int32),
        pltpu.VMEM((NW + 16,), jnp.int32),
        pltpu.VMEM((NW + 16,), jnp.int32),
        pltpu.VMEM((NW + 16,), jnp.int32),
        pltpu.VMEM((NW + 16,), jnp.int32),
        pltpu.VMEM((16, 16), jnp.int32),
        pltpu.VMEM((NODES_PER_W,), jnp.int32),
    ],
)
def _build(ra_hbm, ca_hbm, sta_hbm, nca_hbm, rb_hbm, cb_hbm, stb_hbm,
           ncb_hbm, table_hbm, deg_hbm,
           blk_v, rv, cv, sta_v, nca_v, stb_v, ncb_v, tmp_v, deg_v):
    w = _wid()
    base = pl.multiple_of(w * NODES_PER_W, 8)
    pltpu.sync_copy(sta_hbm, sta_v.at[pl.ds(0, NW)])
    pltpu.sync_copy(nca_hbm, nca_v.at[pl.ds(0, NW)])
    pltpu.sync_copy(stb_hbm, stb_v.at[pl.ds(0, NW)])
    pltpu.sync_copy(ncb_hbm, ncb_v.at[pl.ds(0, NW)])

    def zrow(r, _):
        for g in range(WORDS_PAD // 16):
            blk_v[r, pl.ds(g * 16, 16)] = jnp.zeros((16,), jnp.int32)
        return 0

    lax.fori_loop(0, NODES_PER_W, zrow, 0)

    lanes = lax.iota(jnp.int32, 16)

    def do_pass(r_hbm, c_hbm, s8, nch):
        def chunk(k, _):
            off = pl.multiple_of(
                jnp.minimum(s8 + k * EC_S, E_PAD - EC_S), 8)
            pltpu.sync_copy(r_hbm.at[pl.ds(off, EC_S)], rv.at[pl.ds(0, EC_S)])
            pltpu.sync_copy(c_hbm.at[pl.ds(off, EC_S)], cv.at[pl.ds(0, EC_S)])

            def edge(j, _):
                r = rv[pl.ds(j, 16)][0]
                c = cv[pl.ds(j, 16)][0]
                rl = r - base

                @pl.when((rl >= 0) & (rl < NODES_PER_W))
                def _():
                    g = lax.shift_right_logical(c, jnp.int32(9))
                    lane = lax.shift_right_logical(c, jnp.int32(5)) & 15
                    bitv = lax.shift_left(jnp.int32(1), c & 31)
                    vec = blk_v[rl, pl.ds(g * 16, 16)]
                    blk_v[rl, pl.ds(g * 16, 16)] = vec | jnp.where(
                        lanes == lane, bitv, 0)

                return 0

            return lax.fori_loop(0, EC_S, edge, 0)

        lax.fori_loop(0, nch, chunk, 0)

    do_pass(ra_hbm, ca_hbm,
            sta_v[pl.ds(w, 16)][0], nca_v[pl.ds(w, 16)][0])
    do_pass(rb_hbm, cb_hbm,
            stb_v[pl.ds(w, 16)][0], ncb_v[pl.ds(w, 16)][0])

    def degchunk(i, _):
        def row(j, _):
            acc = jnp.zeros((16,), jnp.int32)
            for t in range(WORDS // 16):
                acc = acc + _popcount(blk_v[i * 16 + j, pl.ds(t * 16, 16)])
            tmp_v[j, :] = acc
            return 0

        lax.fori_loop(0, 16, row, 0)
        deg_v[pl.ds(i * 16, 16)] = _hsum16(tmp_v)
        return 0

    lax.fori_loop(0, NODES_PER_W // 16, degchunk, 0)
    pltpu.sync_copy(blk_v, table_hbm.at[pl.ds(base, NODES_PER_W), :])
    pltpu.sync_copy(deg_v, deg_hbm.at[pl.ds(base, NODES_PER_W)])


def _hsum16(buf_ref):
    rows = lax.iota(jnp.int32, 16)
    tot = jnp.zeros((16,), jnp.int32)
    for t in range(16):
        cols = jnp.full((16,), t, jnp.int32)
        tot = tot + plsc.load_gather(buf_ref, [rows, cols])
    return tot


@functools.partial(
    pl.kernel,
    out_type=jax.ShapeDtypeStruct((E_PAD,), jnp.float32),
    mesh=_mesh,
    compiler_params=_sc_params,
    scratch_types=[
        pltpu.VMEM((EC_N,), jnp.int32),
        pltpu.VMEM((EC_N,), jnp.int32),
        pltpu.VMEM((EC_N, WORDS_PAD), jnp.int32),
        pltpu.VMEM((EC_N, WORDS_PAD), jnp.int32),
        pltpu.VMEM((NPAD,), jnp.int32),
        pltpu.VMEM((EC_N, 16), jnp.int32),
        pltpu.VMEM((EC_N,), jnp.float32),
        pltpu.SemaphoreType.DMA,
        pltpu.SemaphoreType.DMA,
    ],
)
def _norm(table_hbm, src_hbm, dst_hbm, deg_hbm, norm_hbm,
          su_v, sv_v, ur_v, vr_v, deg_v, int_v, nrm_v, sem_u, sem_v):
    w = _wid()
    ebase = w * (E_PAD // NW)
    pltpu.sync_copy(deg_hbm, deg_v)

    def chunk(k, _):
        off = pl.multiple_of(ebase + k * EC_N, 8)
        pltpu.sync_copy(src_hbm.at[pl.ds(off, EC_N)], su_v)
        pltpu.sync_copy(dst_hbm.at[pl.ds(off, EC_N)], sv_v)
        cu = pltpu.async_copy(table_hbm.at[su_v], ur_v, sem_u)
        cv = pltpu.async_copy(table_hbm.at[sv_v], vr_v, sem_v)
        cu.wait()
        cv.wait()

        def edge(j, _):
            acc = jnp.zeros((16,), jnp.int32)
            for t in range(WORDS // 16):
                uw = ur_v[j, pl.ds(t * 16, 16)]
                vw = vr_v[j, pl.ds(t * 16, 16)]
                acc = acc + _popcount(uw & vw)
            int_v[j, :] = acc
            return 0

        lax.fori_loop(0, EC_N, edge, 0)

        for g in range(EC_N // 16):
            rows = lax.iota(jnp.int32, 16) + jnp.int32(g * 16)
            it = jnp.zeros((16,), jnp.int32)
            for t in range(16):
                cols = jnp.full((16,), t, jnp.int32)
                it = it + plsc.load_gather(int_v, [rows, cols])
            iu = su_v[pl.ds(g * 16, 16)]
            iv = sv_v[pl.ds(g * 16, 16)]
            du = plsc.load_gather(deg_v, [iu])
            dv = plsc.load_gather(deg_v, [iv])
            un = du + dv - it
            nrm_v[pl.ds(g * 16, 16)] = (
                it.astype(jnp.float32) / un.astype(jnp.float32))
        pltpu.sync_copy(nrm_v, norm_hbm.at[pl.ds(off, EC_N)])
        return 0

    lax.fori_loop(0, NCH_N, chunk, 0)


@functools.partial(
    pl.kernel,
    out_type=jax.ShapeDtypeStruct((NPAD, D), jnp.float32),
    mesh=_mesh,
    compiler_params=_sc_params,
    scratch_types=[
        pltpu.VMEM((NW + 16,), jnp.int32),
        pltpu.VMEM((NW + 16,), jnp.int32),
        pltpu.VMEM((EC_S,), jnp.int32),
        pltpu.VMEM((EC_S + 16,), jnp.int32),
        pltpu.VMEM((EC_S + 16,), jnp.float32),
        pltpu.VMEM((EC_S, D), jnp.float32),
        pltpu.VMEM((NODES_PER_W, D), jnp.float32),
        pltpu.VMEM((D,), jnp.float32),
        pltpu.SemaphoreType.DMA,
    ],
)
def _segmax(h_hbm, src_hbm, dst_hbm, norm_hbm, starts_hbm, nch_hbm, b_hbm,
            out_hbm, st_v, nc_v, su_v, sd_v, nr_v, rows_v, acc_v, b_v, sem):
    w = _wid()
    base = pl.multiple_of(w * NODES_PER_W, 8)
    pltpu.sync_copy(starts_hbm, st_v.at[pl.ds(0, NW)])
    pltpu.sync_copy(nch_hbm, nc_v.at[pl.ds(0, NW)])
    pltpu.sync_copy(b_hbm, b_v)

    def initrow(r, _):
        for t in range(D // 16):
            acc_v[r, pl.ds(t * 16, 16)] = jnp.full((16,), -3e38, jnp.float32)
        return 0

    lax.fori_loop(0, NODES_PER_W, initrow, 0)

    s8 = st_v[pl.ds(w, 16)][0]
    nch = nc_v[pl.ds(w, 16)][0]

    def chunk(k, _):
        off = pl.multiple_of(jnp.minimum(s8 + k * EC_S, E_PAD - EC_S), 8)
        pltpu.sync_copy(src_hbm.at[pl.ds(off, EC_S)], su_v)
        pltpu.sync_copy(dst_hbm.at[pl.ds(off, EC_S)], sd_v.at[pl.ds(0, EC_S)])
        pltpu.sync_copy(norm_hbm.at[pl.ds(off, EC_S)], nr_v.at[pl.ds(0, EC_S)])
        pltpu.async_copy(h_hbm.at[su_v], rows_v, sem).wait()

        def edge(j, _):
            d = sd_v[pl.ds(j, 16)][0]
            dl = d - base

            @pl.when((dl >= 0) & (dl < NODES_PER_W))
            def _():
                nrm = nr_v[pl.ds(j, 16)][0]
                for t in range(D // 16):
                    msg = rows_v[j, pl.ds(t * 16, 16)] * nrm
                    cur = acc_v[dl, pl.ds(t * 16, 16)]
                    acc_v[dl, pl.ds(t * 16, 16)] = jnp.maximum(cur, msg)

            return 0

        return lax.fori_loop(0, EC_S, edge, 0)

    lax.fori_loop(0, nch, chunk, 0)

    def finrow(r, _):
        for t in range(D // 16):
            a = acc_v[r, pl.ds(t * 16, 16)] + b_v[pl.ds(t * 16, 16)]
            acc_v[r, pl.ds(t * 16, 16)] = jnp.maximum(a, 0.0)
        return 0

    lax.fori_loop(0, NODES_PER_W, finrow, 0)
    pltpu.sync_copy(acc_v, out_hbm.at[pl.ds(base, NODES_PER_W), :])


def _mm_body(x_ref, w_ref, o_ref):
    o_ref[...] = lax.dot_general(
        x_ref[...], w_ref[...], (((1,), (1,)), ((), ())),
        preferred_element_type=jnp.float32)


def _matmul_wt(x, W):
    M, K = x.shape
    Nout = W.shape[0]
    BM = 1024
    return pl.pallas_call(
        _mm_body,
        grid=(M // BM,),
        in_specs=[pl.BlockSpec((BM, K), lambda i: (i, 0)),
                  pl.BlockSpec((Nout, K), lambda i: (0, 0))],
        out_specs=pl.BlockSpec((BM, Nout), lambda i: (i, 0)),
        out_shape=jax.ShapeDtypeStruct((M, Nout), jnp.float32),
    )(x, W)


def _head_body(h_ref, w_ref, b_ref, o_ref):
    logits = lax.dot_general(
        h_ref[...], w_ref[...], (((1,), (1,)), ((), ())),
        preferred_element_type=jnp.float32) + b_ref[...]
    m = jnp.max(logits, axis=-1, keepdims=True)
    ls = logits - m
    o_ref[...] = ls - jnp.log(jnp.sum(jnp.exp(ls), axis=-1, keepdims=True))


def _head(h, Wout, bout):
    M, K = h.shape
    BM = 1024
    return pl.pallas_call(
        _head_body,
        grid=(M // BM,),
        in_specs=[pl.BlockSpec((BM, K), lambda i: (i, 0)),
                  pl.BlockSpec((NCLS, K), lambda i: (0, 0)),
                  pl.BlockSpec((1, NCLS), lambda i: (0, 0))],
        out_specs=pl.BlockSpec((BM, NCLS), lambda i: (i, 0)),
        out_shape=jax.ShapeDtypeStruct((M, NCLS), jnp.float32),
    )(h, Wout, bout.reshape(1, NCLS))


def kernel(x, edge_index, W0, b0, W1, b1, Wout, bout):
    self_loops = jnp.arange(N, dtype=jnp.int32)
    src0 = jnp.concatenate([edge_index[0].astype(jnp.int32), self_loops])
    dst0 = jnp.concatenate([edge_index[1].astype(jnp.int32), self_loops])

    npad_e = E_PAD - E_REAL
    grid = jnp.arange(NW + 1, dtype=jnp.int32) * NODES_PER_W

    def sorted_bounds(key, val, padkey, padval):
        keys = jnp.concatenate([key, jnp.full((npad_e,), padkey, jnp.int32)])
        vals = jnp.concatenate([val, jnp.full((npad_e,), padval, jnp.int32)])
        order = jnp.argsort(keys)
        ks = jnp.take(keys, order).astype(jnp.int32)
        vs = jnp.take(vals, order).astype(jnp.int32)
        bnd = jnp.searchsorted(ks, grid).astype(jnp.int32)
        s8 = bnd[:-1] & jnp.int32(~7)
        cnt = (bnd[1:] - s8 + (EC_S - 1)) // EC_S
        return ks, vs, s8, cnt

    dst_s, src_s, starts8, nch = sorted_bounds(dst0, src0, NPAD - 1, 0)
    srcb_s, dstb_s, starts8b, nchb = sorted_bounds(src0, dst0, NPAD - 1, 0)

    table, deg = _build(dst_s, src_s, starts8, nch,
                        srcb_s, dstb_s, starts8b, nchb)
    norm = _norm(table, src_s, dst_s, deg)

    xp = jnp.zeros((NPAD, D), jnp.float32).at[:N].set(x)
    h = _matmul_wt(xp, W0)
    h = _segmax(h, src_s, dst_s, norm, starts8, nch, b0)
    h = _matmul_wt(h, W1)
    h = _segmax(h, src_s, dst_s, norm, starts8, nch, b1)
    out = _head(h, Wout, bout)
    return out[:N]


# speedup vs baseline: 14.0295x; 1.0066x over previous
import functools

import jax
import jax.numpy as jnp
import numpy as np
from jax import lax
from jax.experimental import pallas as pl
from jax.experimental.pallas import tpu as pltpu
from jax.experimental.pallas import tpu_sc as plsc

N = 10000
NPAD = 10240
NW = 32
NODES_PER_W = NPAD // NW
WORDS = NPAD // 32
WORDS_PAD = 384
E_REAL = 320000 + N
E_PAD = 331776
EC_N = 64
NCH_N = (E_PAD // NW) // EC_N
EC_S = 128
D = 128
NCLS = 64

_mesh = plsc.VectorSubcoreMesh(
    core_axis_name="c", subcore_axis_name="s", num_cores=2, num_subcores=16)
_sc_params = pltpu.CompilerParams(needs_layout_passes=False)


def _wid():
    return lax.axis_index("s") * 2 + lax.axis_index("c")


def _pc_bytes(w):
    c55 = jnp.int32(0x55555555)
    c33 = jnp.int32(0x33333333)
    c0f = jnp.int32(0x0F0F0F0F)
    one = jnp.int32(1)
    w = w - (lax.shift_right_logical(w, one) & c55)
    w = (w & c33) + (lax.shift_right_logical(w, one + one) & c33)
    return (w + lax.shift_right_logical(w, jnp.int32(4))) & c0f


def _pc_fold(acc):
    cmask = jnp.int32(0x00FF00FF)
    h = (acc & cmask) + (lax.shift_right_logical(acc, jnp.int32(8)) & cmask)
    return lax.shift_right_logical(h * jnp.int32(0x00010001), jnp.int32(16))


@functools.partial(
    pl.kernel,
    out_type=[
        jax.ShapeDtypeStruct((NPAD, WORDS_PAD), jnp.int32),
        jax.ShapeDtypeStruct((NPAD,), jnp.int32),
    ],
    mesh=_mesh,
    compiler_params=_sc_params,
    scratch_types=[
        pltpu.VMEM((NODES_PER_W + 1, WORDS_PAD), jnp.int32),
        pltpu.VMEM((EC_S + 16,), jnp.int32),
        pltpu.VMEM((EC_S + 16,), jnp.int32),
        pltpu.VMEM((NW + 16,), jnp.int32),
        pltpu.VMEM((NW + 16,), jnp.int32),
        pltpu.VMEM((NW + 16,), jnp.int32),
        pltpu.VMEM((NW + 16,), jnp.int32),
        pltpu.VMEM((16, 16), jnp.int32),
        pltpu.VMEM((NODES_PER_W,), jnp.int32),
    ],
)
def _build(ra_hbm, ca_hbm, sta_hbm, nca_hbm, rb_hbm, cb_hbm, stb_hbm,
           ncb_hbm, table_hbm, deg_hbm,
           blk_v, rv, cv, sta_v, nca_v, stb_v, ncb_v, tmp_v, deg_v):
    w = _wid()
    base = pl.multiple_of(w * NODES_PER_W, 8)
    pltpu.sync_copy(sta_hbm, sta_v.at[pl.ds(0, NW)])
    pltpu.sync_copy(nca_hbm, nca_v.at[pl.ds(0, NW)])
    pltpu.sync_copy(stb_hbm, stb_v.at[pl.ds(0, NW)])
    pltpu.sync_copy(ncb_hbm, ncb_v.at[pl.ds(0, NW)])

    def zrow(r, _):
        for g in range(WORDS_PAD // 16):
            blk_v[r, pl.ds(g * 16, 16)] = jnp.zeros((16,), jnp.int32)
        return 0

    lax.fori_loop(0, NODES_PER_W, zrow, 0)

    lanes = lax.iota(jnp.int32, 16)

    def do_pass(r_hbm, c_hbm, s8, nch):
        def chunk(k, _):
            off = pl.multiple_of(
                jnp.minimum(s8 + k * EC_S, E_PAD - EC_S), 8)
            pltpu.sync_copy(r_hbm.at[pl.ds(off, EC_S)], rv.at[pl.ds(0, EC_S)])
            pltpu.sync_copy(c_hbm.at[pl.ds(off, EC_S)], cv.at[pl.ds(0, EC_S)])

            def edge(j, _):
                r = rv[pl.ds(j, 16)][0]
                c = cv[pl.ds(j, 16)][0]
                d = r - base
                rl = jnp.where((d >= 0) & (d < NODES_PER_W), d, NODES_PER_W)
                g = lax.shift_right_logical(c, jnp.int32(9))
                lane = lax.shift_right_logical(c, jnp.int32(5)) & 15
                bitv = lax.shift_left(jnp.int32(1), c & 31)
                vec = blk_v[rl, pl.ds(g * 16, 16)]
                blk_v[rl, pl.ds(g * 16, 16)] = vec | jnp.where(
                    lanes == lane, bitv, 0)
                return 0

            return lax.fori_loop(0, EC_S, edge, 0)

        lax.fori_loop(0, nch, chunk, 0)

    do_pass(ra_hbm, ca_hbm,
            sta_v[pl.ds(w, 16)][0], nca_v[pl.ds(w, 16)][0])
    do_pass(rb_hbm, cb_hbm,
            stb_v[pl.ds(w, 16)][0], ncb_v[pl.ds(w, 16)][0])

    def degchunk(i, _):
        def row(j, _):
            acc = jnp.zeros((16,), jnp.int32)
            for t in range(WORDS // 16):
                acc = acc + _pc_bytes(blk_v[i * 16 + j, pl.ds(t * 16, 16)])
            tmp_v[j, :] = _pc_fold(acc)
            return 0

        lax.fori_loop(0, 16, row, 0)
        deg_v[pl.ds(i * 16, 16)] = _hsum16(tmp_v)
        return 0

    lax.fori_loop(0, NODES_PER_W // 16, degchunk, 0)
    pltpu.sync_copy(blk_v.at[pl.ds(0, NODES_PER_W), :],
                    table_hbm.at[pl.ds(base, NODES_PER_W), :])
    pltpu.sync_copy(deg_v, deg_hbm.at[pl.ds(base, NODES_PER_W)])


def _hsum16(buf_ref):
    rows = lax.iota(jnp.int32, 16)
    tot = jnp.zeros((16,), jnp.int32)
    for t in range(16):
        cols = jnp.full((16,), t, jnp.int32)
        tot = tot + plsc.load_gather(buf_ref, [rows, cols])
    return tot


@functools.partial(
    pl.kernel,
    out_type=jax.ShapeDtypeStruct((E_PAD,), jnp.float32),
    mesh=_mesh,
    compiler_params=_sc_params,
    scratch_types=[
        pltpu.VMEM((EC_N,), jnp.int32),
        pltpu.VMEM((EC_N,), jnp.int32),
        pltpu.VMEM((EC_N, WORDS_PAD), jnp.int32),
        pltpu.VMEM((EC_N, WORDS_PAD), jnp.int32),
        pltpu.VMEM((NPAD,), jnp.int32),
        pltpu.VMEM((EC_N, 16), jnp.int32),
        pltpu.VMEM((EC_N,), jnp.float32),
        pltpu.SemaphoreType.DMA,
        pltpu.SemaphoreType.DMA,
    ],
)
def _norm(table_hbm, src_hbm, dst_hbm, deg_hbm, norm_hbm,
          su_v, sv_v, ur_v, vr_v, deg_v, int_v, nrm_v, sem_u, sem_v):
    w = _wid()
    ebase = w * (E_PAD // NW)
    pltpu.sync_copy(deg_hbm, deg_v)

    def chunk(k, _):
        off = pl.multiple_of(ebase + k * EC_N, 8)
        pltpu.sync_copy(src_hbm.at[pl.ds(off, EC_N)], su_v)
        pltpu.sync_copy(dst_hbm.at[pl.ds(off, EC_N)], sv_v)
        cu = pltpu.async_copy(table_hbm.at[su_v], ur_v, sem_u)
        cv = pltpu.async_copy(table_hbm.at[sv_v], vr_v, sem_v)
        cu.wait()
        cv.wait()

        def edge(j, _):
            acc = jnp.zeros((16,), jnp.int32)
            for t in range(WORDS // 16):
                uw = ur_v[j, pl.ds(t * 16, 16)]
                vw = vr_v[j, pl.ds(t * 16, 16)]
                acc = acc + _pc_bytes(uw & vw)
            int_v[j, :] = _pc_fold(acc)
            return 0

        lax.fori_loop(0, EC_N, edge, 0)

        for g in range(EC_N // 16):
            rows = lax.iota(jnp.int32, 16) + jnp.int32(g * 16)
            it = jnp.zeros((16,), jnp.int32)
            for t in range(16):
                cols = jnp.full((16,), t, jnp.int32)
                it = it + plsc.load_gather(int_v, [rows, cols])
            iu = su_v[pl.ds(g * 16, 16)]
            iv = sv_v[pl.ds(g * 16, 16)]
            du = plsc.load_gather(deg_v, [iu])
            dv = plsc.load_gather(deg_v, [iv])
            un = du + dv - it
            nrm_v[pl.ds(g * 16, 16)] = (
                it.astype(jnp.float32) / un.astype(jnp.float32))
        pltpu.sync_copy(nrm_v, norm_hbm.at[pl.ds(off, EC_N)])
        return 0

    lax.fori_loop(0, NCH_N, chunk, 0)


@functools.partial(
    pl.kernel,
    out_type=jax.ShapeDtypeStruct((NPAD, D), jnp.float32),
    mesh=_mesh,
    compiler_params=_sc_params,
    scratch_types=[
        pltpu.VMEM((NW + 16,), jnp.int32),
        pltpu.VMEM((NW + 16,), jnp.int32),
        pltpu.VMEM((EC_S,), jnp.int32),
        pltpu.VMEM((EC_S + 16,), jnp.int32),
        pltpu.VMEM((EC_S + 16,), jnp.float32),
        pltpu.VMEM((EC_S, D), jnp.float32),
        pltpu.VMEM((NODES_PER_W + 1, D), jnp.float32),
        pltpu.VMEM((D,), jnp.float32),
        pltpu.SemaphoreType.DMA,
    ],
)
def _segmax(h_hbm, src_hbm, dst_hbm, norm_hbm, starts_hbm, nch_hbm, b_hbm,
            out_hbm, st_v, nc_v, su_v, sd_v, nr_v, rows_v, acc_v, b_v, sem):
    w = _wid()
    base = pl.multiple_of(w * NODES_PER_W, 8)
    pltpu.sync_copy(starts_hbm, st_v.at[pl.ds(0, NW)])
    pltpu.sync_copy(nch_hbm, nc_v.at[pl.ds(0, NW)])
    pltpu.sync_copy(b_hbm, b_v)

    def initrow(r, _):
        for t in range(D // 16):
            acc_v[r, pl.ds(t * 16, 16)] = jnp.full((16,), -3e38, jnp.float32)
        return 0

    lax.fori_loop(0, NODES_PER_W, initrow, 0)

    s8 = st_v[pl.ds(w, 16)][0]
    nch = nc_v[pl.ds(w, 16)][0]

    def chunk(k, _):
        off = pl.multiple_of(jnp.minimum(s8 + k * EC_S, E_PAD - EC_S), 8)
        pltpu.sync_copy(src_hbm.at[pl.ds(off, EC_S)], su_v)
        pltpu.sync_copy(dst_hbm.at[pl.ds(off, EC_S)], sd_v.at[pl.ds(0, EC_S)])
        pltpu.sync_copy(norm_hbm.at[pl.ds(off, EC_S)], nr_v.at[pl.ds(0, EC_S)])
        pltpu.async_copy(h_hbm.at[su_v], rows_v, sem).wait()

        def edge(j, _):
            d = sd_v[pl.ds(j, 16)][0] - base
            dl = jnp.where((d >= 0) & (d < NODES_PER_W), d, NODES_PER_W)
            nrm = nr_v[pl.ds(j, 16)][0]
            for t in range(D // 16):
                msg = rows_v[j, pl.ds(t * 16, 16)] * nrm
                cur = acc_v[dl, pl.ds(t * 16, 16)]
                acc_v[dl, pl.ds(t * 16, 16)] = jnp.maximum(cur, msg)
            return 0

        return lax.fori_loop(0, EC_S, edge, 0)

    lax.fori_loop(0, nch, chunk, 0)

    def finrow(r, _):
        for t in range(D // 16):
            a = acc_v[r, pl.ds(t * 16, 16)] + b_v[pl.ds(t * 16, 16)]
            acc_v[r, pl.ds(t * 16, 16)] = jnp.maximum(a, 0.0)
        return 0

    lax.fori_loop(0, NODES_PER_W, finrow, 0)
    pltpu.sync_copy(acc_v.at[pl.ds(0, NODES_PER_W), :],
                    out_hbm.at[pl.ds(base, NODES_PER_W), :])


def _mm_body(x_ref, w_ref, o_ref):
    o_ref[...] = lax.dot_general(
        x_ref[...], w_ref[...], (((1,), (1,)), ((), ())),
        preferred_element_type=jnp.float32)


def _matmul_wt(x, W):
    M, K = x.shape
    Nout = W.shape[0]
    BM = 1024
    return pl.pallas_call(
        _mm_body,
        grid=(M // BM,),
        in_specs=[pl.BlockSpec((BM, K), lambda i: (i, 0)),
                  pl.BlockSpec((Nout, K), lambda i: (0, 0))],
        out_specs=pl.BlockSpec((BM, Nout), lambda i: (i, 0)),
        out_shape=jax.ShapeDtypeStruct((M, Nout), jnp.float32),
    )(x, W)


def _head_body(h_ref, w_ref, b_ref, o_ref):
    logits = lax.dot_general(
        h_ref[...], w_ref[...], (((1,), (1,)), ((), ())),
        preferred_element_type=jnp.float32) + b_ref[...]
    m = jnp.max(logits, axis=-1, keepdims=True)
    ls = logits - m
    o_ref[...] = ls - jnp.log(jnp.sum(jnp.exp(ls), axis=-1, keepdims=True))


def _head(h, Wout, bout):
    M, K = h.shape
    BM = 1024
    return pl.pallas_call(
        _head_body,
        grid=(M // BM,),
        in_specs=[pl.BlockSpec((BM, K), lambda i: (i, 0)),
                  pl.BlockSpec((NCLS, K), lambda i: (0, 0)),
                  pl.BlockSpec((1, NCLS), lambda i: (0, 0))],
        out_specs=pl.BlockSpec((BM, NCLS), lambda i: (i, 0)),
        out_shape=jax.ShapeDtypeStruct((M, NCLS), jnp.float32),
    )(h, Wout, bout.reshape(1, NCLS))


def kernel(x, edge_index, W0, b0, W1, b1, Wout, bout):
    self_loops = jnp.arange(N, dtype=jnp.int32)
    src0 = jnp.concatenate([edge_index[0].astype(jnp.int32), self_loops])
    dst0 = jnp.concatenate([edge_index[1].astype(jnp.int32), self_loops])

    npad_e = E_PAD - E_REAL
    grid = jnp.arange(NW + 1, dtype=jnp.int32) * NODES_PER_W

    def sorted_bounds(key, val, padkey, padval):
        keys = jnp.concatenate([key, jnp.full((npad_e,), padkey, jnp.int32)])
        vals = jnp.concatenate([val, jnp.full((npad_e,), padval, jnp.int32)])
        order = jnp.argsort(keys)
        ks = jnp.take(keys, order).astype(jnp.int32)
        vs = jnp.take(vals, order).astype(jnp.int32)
        bnd = jnp.searchsorted(ks, grid).astype(jnp.int32)
        s8 = bnd[:-1] & jnp.int32(~7)
        cnt = (bnd[1:] - s8 + (EC_S - 1)) // EC_S
        return ks, vs, s8, cnt

    dst_s, src_s, starts8, nch = sorted_bounds(dst0, src0, NPAD - 1, 0)
    srcb_s, dstb_s, starts8b, nchb = sorted_bounds(src0, dst0, NPAD - 1, 0)

    table, deg = _build(dst_s, src_s, starts8, nch,
                        srcb_s, dstb_s, starts8b, nchb)
    norm = _norm(table, src_s, dst_s, deg)

    xp = jnp.zeros((NPAD, D), jnp.float32).at[:N].set(x)
    h = _matmul_wt(xp, W0)
    h = _segmax(h, src_s, dst_s, norm, starts8, nch, b0)
    h = _matmul_wt(h, W1)
    h = _segmax(h, src_s, dst_s, norm, starts8, nch, b1)
    out = _head(h, Wout, bout)
    return out[:N]
